# Initial kernel scaffold; baseline (speedup 1.0000x reference)
import jax
import jax.numpy as jnp
from jax.experimental import pallas as pl


def kernel(x, table):
    raise NotImplementedError("write your pallas kernel here")


# speedup vs baseline: 1.8195x; 1.8195x over previous
import functools

import jax
import jax.numpy as jnp
from jax import lax
from jax.experimental import pallas as pl
from jax.experimental.pallas import tpu as pltpu
from jax.experimental.pallas import tpu_sc as plsc

_LATENT = 3
_D = _LATENT * 64 * 64
_B = 4096
_NC = 2
_NS = 16
_NW = _NC * _NS
_BPW = _B // _NW
_CH = 4
_NCHUNK = _BPW // _CH


def _build_gather():
    mesh = plsc.VectorSubcoreMesh(core_axis_name="c", subcore_axis_name="s")

    @functools.partial(
        pl.kernel,
        mesh=mesh,
        out_type=jax.ShapeDtypeStruct((_B, _D), jnp.float32),
        scratch_types=[
            pltpu.VMEM((_NCHUNK, _CH), jnp.int32),
            pltpu.VMEM((2, _CH, _D), jnp.float32),
            pltpu.SemaphoreType.DMA,
        ],
    )
    def gather(idx_hbm, table_hbm, out_hbm, idx_v, rows_v, gsem):
        wid = lax.axis_index("s") * _NC + lax.axis_index("c")
        base = wid * _BPW
        pltpu.sync_copy(idx_hbm.at[wid], idx_v)
        pltpu.async_copy(table_hbm.at[idx_v.at[0]], rows_v.at[0], gsem)

        def body(g, carry):
            buf = lax.rem(g, 2)
            pltpu.make_async_copy(
                table_hbm.at[idx_v.at[g]], rows_v.at[buf], gsem
            ).wait()

            @pl.when(g + 1 < _NCHUNK)
            def _start_next():
                pltpu.async_copy(
                    table_hbm.at[idx_v.at[g + 1]], rows_v.at[1 - buf], gsem
                )

            pltpu.sync_copy(
                rows_v.at[buf], out_hbm.at[pl.ds(base + g * _CH, _CH)]
            )
            return carry

        lax.fori_loop(0, _NCHUNK, body, 0)

    return gather


_GATHER = _build_gather()


def kernel(x, table):
    idx = x.astype(jnp.int32).reshape(_NW, _NCHUNK, _CH)
    out = _GATHER(idx, table)
    return out.reshape(-1, _LATENT, 64, 64)
